# baseline (speedup 1.0000x reference)
import functools

import jax
import jax.numpy as jnp
from jax import lax
from jax.experimental import pallas as pl
from jax.experimental.pallas import tpu as pltpu
from jax.experimental.pallas import tpu_sc as plsc

NU = 50000
NUP = 50048
D = 64
H = 32
NE = 800000
B = 4096
NS = 16
NLAYERS = 3

EPC = NE // NS
CH = 400
NCH = EPC // CH
ZPR = NUP // NS
ZR = 184
BPC = B // NS
BCH = 64
MCH = 391


def _sc_body(u0f, i0f, relf, e0s, e0d, w0r, e1s, e1d, w1r, uir, iir,
             u1f, i1f, u2f, i2f, u3f, i3f, ugr, igr,
             u0m0, u0m1, i0m0, i0m1, u1m0, u1m1, i1m0, i1m1,
             u2m0, u2m1, i2m0, i2m1,
             acc, gixA, sixA, wvA, rowsA, gixB, sixB, wvB, rowsB,
             rv, bidx, semA, semB):
    core = lax.axis_index("c")
    sub = lax.axis_index("s")
    coff = core * NUP

    pltpu.sync_copy(relf.at[pl.ds(core * 8, 8)], rv)

    zero16 = jnp.zeros((16,), jnp.float32)

    def zero_acc():
        @pl.loop(0, ZR)
        def _(r):
            rowsB[r, pl.ds(0, 16)] = zero16
            rowsB[r, pl.ds(16, 16)] = zero16

        base = sub * ZPR

        @pl.loop(0, ZPR, step=ZR)
        def _(r):
            pltpu.sync_copy(rowsB.at[pl.ds(0, ZR)],
                            acc.at[pl.ds(base + r, ZR)])

    def edge_op(gidx_hbm, sidx_hbm, w_hbm, table_hbm):
        ebase = sub * EPC

        def prefetch(c, gix, six, wvb, rows, sem):
            st = ebase + c * CH
            pltpu.sync_copy(gidx_hbm.at[pl.ds(st, CH)], gix)
            pltpu.sync_copy(sidx_hbm.at[pl.ds(st, CH)], six)
            pltpu.sync_copy(w_hbm.at[pl.ds(st, CH)], wvb)

            @pl.loop(0, CH, step=16)
            def _(j):
                gix[pl.ds(j, 16)] = gix[pl.ds(j, 16)] + coff

            pltpu.async_copy(table_hbm.at[gix], rows, sem)

        def compute(gix, six, wvb, rows, sem):
            pltpu.make_async_copy(table_hbm.at[gix], rows, sem).wait()

            @pl.loop(0, CH, step=16)
            def _(g):
                w16 = wvb[pl.ds(g, 16)]
                gdn = lax.GatherDimensionNumbers(
                    offset_dims=(), collapsed_slice_dims=(0,),
                    start_index_map=(0,))
                for k in range(16):
                    idx = jnp.full((16, 1), k, jnp.int32)
                    wk = lax.gather(
                        w16, idx, gdn, (1,),
                        mode=lax.GatherScatterMode.PROMISE_IN_BOUNDS)
                    rows[g + k, pl.ds(0, 16)] = rows[g + k, pl.ds(0, 16)] * wk
                    rows[g + k, pl.ds(16, 16)] = (rows[g + k, pl.ds(16, 16)]
                                                  * wk)

            pltpu.sync_copy(rows, acc.at[six], add=True)

        prefetch(0, gixA, sixA, wvA, rowsA, semA)

        @pl.loop(0, NCH)
        def _(c):
            @pl.when(c % 2 == 0)
            def _():
                @pl.when(c + 1 < NCH)
                def _():
                    prefetch(c + 1, gixB, sixB, wvB, rowsB, semB)

                compute(gixA, sixA, wvA, rowsA, semA)

            @pl.when(c % 2 == 1)
            def _():
                @pl.when(c + 1 < NCH)
                def _():
                    prefetch(c + 1, gixA, sixA, wvA, rowsA, semA)

                compute(gixB, sixB, wvB, rowsB, semB)

    def mod_write(src_ref, src_off, m0, m1):
        rv00 = rv[0, pl.ds(0, 16)]
        rv01 = rv[0, pl.ds(16, 16)]
        rv10 = rv[1, pl.ds(0, 16)]
        rv11 = rv[1, pl.ds(16, 16)]
        dbase = coff + sub * ZPR

        @pl.loop(0, ZPR, step=MCH)
        def _(q):
            pltpu.sync_copy(src_ref.at[pl.ds(src_off + q, MCH)],
                            rowsA.at[pl.ds(0, MCH)])

            @pl.loop(0, MCH)
            def _(r):
                a0 = rowsA[r, pl.ds(0, 16)]
                a1 = rowsA[r, pl.ds(16, 16)]
                rowsB[r, pl.ds(0, 16)] = a0 * rv00
                rowsB[r, pl.ds(16, 16)] = a1 * rv01
                rowsA[r, pl.ds(0, 16)] = a0 * rv10
                rowsA[r, pl.ds(16, 16)] = a1 * rv11

            pltpu.sync_copy(rowsB.at[pl.ds(0, MCH)],
                            m0.at[pl.ds(dbase + q, MCH)])
            pltpu.sync_copy(rowsA.at[pl.ds(0, MCH)],
                            m1.at[pl.ds(dbase + q, MCH)])

    def flush(dst_hbm, m0=None, m1=None):
        base = sub * ZPR
        pltpu.sync_copy(acc.at[pl.ds(base, ZPR)],
                        dst_hbm.at[pl.ds(coff + base, ZPR)])
        if m0 is not None:
            mod_write(acc, base, m0, m1)

    mod_write(u0f, coff + sub * ZPR, u0m0, u0m1)
    mod_write(i0f, coff + sub * ZPR, i0m0, i0m1)

    tables = [(u0f, i0f), (u1f, i1f), (u2f, i2f), (u3f, i3f)]
    mods = [(u0m0, u0m1, i0m0, i0m1), (u1m0, u1m1, i1m0, i1m1),
            (u2m0, u2m1, i2m0, i2m1), (None, None, None, None)]
    for layer in range(NLAYERS):
        um0, um1, im0, im1 = mods[layer]
        nxt = mods[layer + 1]
        uout, iout = tables[layer + 1]
        zero_acc()
        plsc.subcore_barrier()
        edge_op(e0d, e0s, w0r, im0)
        edge_op(e1d, e1s, w1r, im1)
        plsc.subcore_barrier()
        flush(uout, nxt[0], nxt[1])
        zero_acc()
        plsc.subcore_barrier()
        edge_op(e0s, e0d, w0r, um0)
        edge_op(e1s, e1d, w1r, um1)
        plsc.subcore_barrier()
        flush(iout, nxt[2], nxt[3])

    plsc.subcore_barrier()

    def gather_sum(idx_hbm, tabs, out_hbm):
        bbase = sub * BPC

        @pl.loop(0, BPC, step=BCH)
        def _(q):
            pltpu.sync_copy(idx_hbm.at[pl.ds(bbase + q, BCH)], bidx)

            @pl.loop(0, BCH, step=16)
            def _(j):
                bidx[pl.ds(j, 16)] = bidx[pl.ds(j, 16)] + coff

            pltpu.async_copy(tabs[0].at[bidx],
                             rowsA.at[pl.ds(0, BCH)], semA).wait()
            for t in tabs[1:]:
                pltpu.async_copy(t.at[bidx],
                                 rowsB.at[pl.ds(0, BCH)], semB).wait()

                @pl.loop(0, BCH)
                def _(k):
                    rowsA[k, pl.ds(0, 16)] = (rowsA[k, pl.ds(0, 16)]
                                              + rowsB[k, pl.ds(0, 16)])
                    rowsA[k, pl.ds(16, 16)] = (rowsA[k, pl.ds(16, 16)]
                                               + rowsB[k, pl.ds(16, 16)])

            pltpu.sync_copy(rowsA.at[pl.ds(0, BCH)],
                            out_hbm.at[pl.ds(core * B + bbase + q, BCH)])

    gather_sum(uir, [u0f, u1f, u2f, u3f], ugr)
    gather_sum(iir, [i0f, i1f, i2f, i3f], igr)


def _dot_body(ug_ref, ig_ref, o_ref):
    prod = ug_ref[...] * ig_ref[...]
    s = prod.sum(axis=1)
    o_ref[0, :] = (s[:B] + s[B:]) * (1.0 / 16.0)


def kernel(user_indices, item_indices, edge_t0_index, edge_t0_weights,
           edge_t1_index, edge_t1_weights, user_emb, item_emb, rel_emb):
    f32 = jnp.float32
    e0s = edge_t0_index[0].astype(jnp.int32)
    e0d = edge_t0_index[1].astype(jnp.int32)
    e1s = edge_t1_index[0].astype(jnp.int32)
    e1d = edge_t1_index[1].astype(jnp.int32)
    w0 = edge_t0_weights.astype(f32)
    w1 = edge_t1_weights.astype(f32)
    ui = user_indices.astype(jnp.int32)
    ii = item_indices.astype(jnp.int32)

    zpad = jnp.zeros((NUP - NU, H), f32)
    u0f = jnp.concatenate([user_emb[:, :H], zpad, user_emb[:, H:], zpad],
                          axis=0)
    i0f = jnp.concatenate([item_emb[:, :H], zpad, item_emb[:, H:], zpad],
                          axis=0)
    relf = jnp.zeros((16, H), f32)
    relf = relf.at[0:2].set(rel_emb[:, :H]).at[8:10].set(rel_emb[:, H:])

    mesh = plsc.VectorSubcoreMesh(core_axis_name="c", subcore_axis_name="s")
    tab = jax.ShapeDtypeStruct((2 * NUP, H), f32)
    gat = jax.ShapeDtypeStruct((2 * B, H), f32)

    sc = pl.kernel(
        _sc_body,
        out_type=(tab, tab, tab, tab, tab, tab, gat, gat,
                  tab, tab, tab, tab, tab, tab, tab, tab,
                  tab, tab, tab, tab),
        mesh=mesh,
        compiler_params=pltpu.CompilerParams(use_tc_tiling_on_sc=False),
        scratch_types=[
            pltpu.VMEM_SHARED((NUP, H), f32),
            pltpu.VMEM((CH,), jnp.int32),
            pltpu.VMEM((CH,), jnp.int32),
            pltpu.VMEM((CH,), f32),
            pltpu.VMEM((CH, H), f32),
            pltpu.VMEM((CH,), jnp.int32),
            pltpu.VMEM((CH,), jnp.int32),
            pltpu.VMEM((CH,), f32),
            pltpu.VMEM((CH, H), f32),
            pltpu.VMEM((8, H), f32),
            pltpu.VMEM((BCH,), jnp.int32),
            pltpu.SemaphoreType.DMA,
            pltpu.SemaphoreType.DMA,
        ],
    )
    outs = sc(u0f, i0f, relf, e0s, e0d, w0, e1s, e1d, w1, ui, ii)
    ug, ig = outs[6], outs[7]

    out = pl.pallas_call(
        _dot_body,
        out_shape=jax.ShapeDtypeStruct((1, B), f32),
    )(ug, ig)
    return out[0]


# speedup vs baseline: 1.0098x; 1.0098x over previous
import functools

import jax
import jax.numpy as jnp
from jax import lax
from jax.experimental import pallas as pl
from jax.experimental.pallas import tpu as pltpu
from jax.experimental.pallas import tpu_sc as plsc

NU = 50000
NUP = 50048
D = 64
H = 32
NE = 800000
B = 4096
NS = 16
NLAYERS = 3

EPC = NE // NS
CH = 400
NCH = EPC // CH
ZPR = NUP // NS
ZR = 184
BPC = B // NS
BCH = 64


def _sc_body(u0f, i0f, relf, e0s, e0d, w0r, e1s, e1d, w1r, uir, iir,
             u1f, i1f, u2f, i2f, u3f, i3f, ugr, igr,
             acc, gixA, sixA, wvA, rowsA, gixB, sixB, wvB, rowsB,
             rv, bidx, semA, semB, ssemA, ssemB):
    core = lax.axis_index("c")
    sub = lax.axis_index("s")
    coff = core * NUP

    pltpu.sync_copy(relf.at[pl.ds(core * 8, 8)], rv)

    zero16 = jnp.zeros((16,), jnp.float32)

    def zero_acc():
        @pl.loop(0, ZR)
        def _(r):
            rowsB[r, pl.ds(0, 16)] = zero16
            rowsB[r, pl.ds(16, 16)] = zero16

        base = sub * ZPR

        @pl.loop(0, ZPR, step=ZR)
        def _(r):
            pltpu.sync_copy(rowsB.at[pl.ds(0, ZR)],
                            acc.at[pl.ds(base + r, ZR)])

    def edge_op(gidx_hbm, sidx_hbm, w_hbm, table_hbm, rsel):
        ebase = sub * EPC
        rv0 = rv[rsel, pl.ds(0, 16)]
        rv1 = rv[rsel, pl.ds(16, 16)]

        def scat_wait(six, rows, ssem):
            pltpu.make_async_copy(rows, acc.at[six], ssem).wait()

        def prefetch(c, gix, six, wvb, rows, sem):
            st = ebase + c * CH
            pltpu.sync_copy(gidx_hbm.at[pl.ds(st, CH)], gix)
            pltpu.sync_copy(sidx_hbm.at[pl.ds(st, CH)], six)
            pltpu.sync_copy(w_hbm.at[pl.ds(st, CH)], wvb)

            @pl.loop(0, CH, step=16)
            def _(j):
                gix[pl.ds(j, 16)] = gix[pl.ds(j, 16)] + coff

            pltpu.async_copy(table_hbm.at[gix], rows, sem)

        def compute(gix, six, wvb, rows, sem, ssem):
            pltpu.make_async_copy(table_hbm.at[gix], rows, sem).wait()

            @pl.loop(0, CH, step=16)
            def _(g):
                w16 = wvb[pl.ds(g, 16)]
                gdn = lax.GatherDimensionNumbers(
                    offset_dims=(), collapsed_slice_dims=(0,),
                    start_index_map=(0,))
                for k in range(16):
                    idx = jnp.full((16, 1), k, jnp.int32)
                    wk = lax.gather(
                        w16, idx, gdn, (1,),
                        mode=lax.GatherScatterMode.PROMISE_IN_BOUNDS)
                    rows[g + k, pl.ds(0, 16)] = (rows[g + k, pl.ds(0, 16)]
                                                 * (wk * rv0))
                    rows[g + k, pl.ds(16, 16)] = (rows[g + k, pl.ds(16, 16)]
                                                  * (wk * rv1))

            pltpu.async_copy(rows, acc.at[six], ssem, add=True)

        prefetch(0, gixA, sixA, wvA, rowsA, semA)

        @pl.loop(0, NCH)
        def _(c):
            @pl.when(c % 2 == 0)
            def _():
                @pl.when(c + 1 < NCH)
                def _():
                    @pl.when(c >= 1)
                    def _():
                        scat_wait(sixB, rowsB, ssemB)

                    prefetch(c + 1, gixB, sixB, wvB, rowsB, semB)

                compute(gixA, sixA, wvA, rowsA, semA, ssemA)

            @pl.when(c % 2 == 1)
            def _():
                @pl.when(c + 1 < NCH)
                def _():
                    scat_wait(sixA, rowsA, ssemA)
                    prefetch(c + 1, gixA, sixA, wvA, rowsA, semA)

                compute(gixB, sixB, wvB, rowsB, semB, ssemB)

        scat_wait(sixB, rowsB, ssemB)
        scat_wait(sixA, rowsA, ssemA)

    def flush(dst_hbm):
        base = sub * ZPR
        pltpu.sync_copy(acc.at[pl.ds(base, ZPR)],
                        dst_hbm.at[pl.ds(coff + base, ZPR)])

    tables = [(u0f, i0f), (u1f, i1f), (u2f, i2f), (u3f, i3f)]
    for layer in range(NLAYERS):
        uin, iin = tables[layer]
        uout, iout = tables[layer + 1]
        zero_acc()
        plsc.subcore_barrier()
        edge_op(e0d, e0s, w0r, iin, 0)
        edge_op(e1d, e1s, w1r, iin, 1)
        plsc.subcore_barrier()
        flush(uout)
        zero_acc()
        plsc.subcore_barrier()
        edge_op(e0s, e0d, w0r, uin, 0)
        edge_op(e1s, e1d, w1r, uin, 1)
        plsc.subcore_barrier()
        flush(iout)

    plsc.subcore_barrier()

    def gather_sum(idx_hbm, tabs, out_hbm):
        bbase = sub * BPC

        @pl.loop(0, BPC, step=BCH)
        def _(q):
            pltpu.sync_copy(idx_hbm.at[pl.ds(bbase + q, BCH)], bidx)

            @pl.loop(0, BCH, step=16)
            def _(j):
                bidx[pl.ds(j, 16)] = bidx[pl.ds(j, 16)] + coff

            pltpu.async_copy(tabs[0].at[bidx],
                             rowsA.at[pl.ds(0, BCH)], semA).wait()
            for t in tabs[1:]:
                pltpu.async_copy(t.at[bidx],
                                 rowsB.at[pl.ds(0, BCH)], semB).wait()

                @pl.loop(0, BCH)
                def _(k):
                    rowsA[k, pl.ds(0, 16)] = (rowsA[k, pl.ds(0, 16)]
                                              + rowsB[k, pl.ds(0, 16)])
                    rowsA[k, pl.ds(16, 16)] = (rowsA[k, pl.ds(16, 16)]
                                               + rowsB[k, pl.ds(16, 16)])

            pltpu.sync_copy(rowsA.at[pl.ds(0, BCH)],
                            out_hbm.at[pl.ds(core * B + bbase + q, BCH)])

    gather_sum(uir, [u0f, u1f, u2f, u3f], ugr)
    gather_sum(iir, [i0f, i1f, i2f, i3f], igr)


def _dot_body(ug_ref, ig_ref, o_ref):
    prod = ug_ref[...] * ig_ref[...]
    s = prod.sum(axis=1)
    o_ref[0, :] = (s[:B] + s[B:]) * (1.0 / 16.0)


def kernel(user_indices, item_indices, edge_t0_index, edge_t0_weights,
           edge_t1_index, edge_t1_weights, user_emb, item_emb, rel_emb):
    f32 = jnp.float32
    e0s = edge_t0_index[0].astype(jnp.int32)
    e0d = edge_t0_index[1].astype(jnp.int32)
    e1s = edge_t1_index[0].astype(jnp.int32)
    e1d = edge_t1_index[1].astype(jnp.int32)
    w0 = edge_t0_weights.astype(f32)
    w1 = edge_t1_weights.astype(f32)
    ui = user_indices.astype(jnp.int32)
    ii = item_indices.astype(jnp.int32)

    zpad = jnp.zeros((NUP - NU, H), f32)
    u0f = jnp.concatenate([user_emb[:, :H], zpad, user_emb[:, H:], zpad],
                          axis=0)
    i0f = jnp.concatenate([item_emb[:, :H], zpad, item_emb[:, H:], zpad],
                          axis=0)
    relf = jnp.zeros((16, H), f32)
    relf = relf.at[0:2].set(rel_emb[:, :H]).at[8:10].set(rel_emb[:, H:])

    mesh = plsc.VectorSubcoreMesh(core_axis_name="c", subcore_axis_name="s")
    tab = jax.ShapeDtypeStruct((2 * NUP, H), f32)
    gat = jax.ShapeDtypeStruct((2 * B, H), f32)

    sc = pl.kernel(
        _sc_body,
        out_type=(tab, tab, tab, tab, tab, tab, gat, gat),
        mesh=mesh,
        compiler_params=pltpu.CompilerParams(use_tc_tiling_on_sc=False),
        scratch_types=[
            pltpu.VMEM_SHARED((NUP, H), f32),
            pltpu.VMEM((CH,), jnp.int32),
            pltpu.VMEM((CH,), jnp.int32),
            pltpu.VMEM((CH,), f32),
            pltpu.VMEM((CH, H), f32),
            pltpu.VMEM((CH,), jnp.int32),
            pltpu.VMEM((CH,), jnp.int32),
            pltpu.VMEM((CH,), f32),
            pltpu.VMEM((CH, H), f32),
            pltpu.VMEM((8, H), f32),
            pltpu.VMEM((BCH,), jnp.int32),
            pltpu.SemaphoreType.DMA,
            pltpu.SemaphoreType.DMA,
            pltpu.SemaphoreType.DMA,
            pltpu.SemaphoreType.DMA,
        ],
    )
    outs = sc(u0f, i0f, relf, e0s, e0d, w0, e1s, e1d, w1, ui, ii)
    ug, ig = outs[6], outs[7]

    out = pl.pallas_call(
        _dot_body,
        out_shape=jax.ShapeDtypeStruct((1, B), f32),
    )(ug, ig)
    return out[0]
